# baseline (speedup 1.0000x reference)
import jax
import jax.numpy as jnp
from jax import lax
from jax.experimental import pallas as pl
from jax.experimental.pallas import tpu as pltpu
from jax.experimental.pallas import tpu_sc as plsc

_N = 10000
_E = 320000
_D = 128
_NP = 10240
_NT = 32
_ETP = 10240
_EP = _NT * _ETP
_K = 40
_EC = 2560
_NC = _ETP // _EC
_GC = _EC // _K
_NB = 4
_CW = _EC
_RB = 1000


def _pq_body(h_ref, w_ref, pq_ref, b_ref):
    h = h_ref[...]
    w = w_ref[...]
    pq = lax.dot_general(h, w, (((1,), (1,)), ((), ())),
                         preferred_element_type=jnp.float32)
    pq_ref[...] = pq
    s = jnp.max(pq[:, 0]) + jnp.max(pq[:, 1])
    b = jnp.where(s >= 0, s, 0.01 * s)
    b_ref[...] = jnp.full((8, 128), b, jnp.float32)


def _sc_body(h_hbm, srcg_hbm, dstg_hbm, p_hbm, q_hbm,
             u_hbm, s_hbm,
             src_c, dst_c, ex_c, p_v, q_v, rows4, szbuf,
             u_sh, s_sh, gsem, ssem):
    cid = lax.axis_index("c")
    sid = lax.axis_index("s")
    wid = cid * 16 + sid
    nbase = sid * 640

    zv = jnp.zeros((16,), jnp.float32)

    def zrow(r, c):
        for c8 in range(8):
            rows4[0, r, pl.ds(16 * c8, 16)] = zv
        return c
    lax.fori_loop(0, 40, zrow, 0)

    def zs(i, c):
        szbuf[pl.ds(16 * i, 16)] = zv
        return c
    lax.fori_loop(0, 5, zs, 0)

    for k in range(16):
        pltpu.sync_copy(rows4.at[0], u_sh.at[pl.ds(nbase + 40 * k, 40)])
    for k in range(8):
        pltpu.sync_copy(szbuf, s_sh.at[pl.ds(nbase + 80 * k, 80)])

    pltpu.sync_copy(p_hbm, p_v)
    pltpu.sync_copy(q_hbm, q_v)

    plsc.subcore_barrier()

    iota = lax.iota(jnp.int32, 16)
    bvec = p_v[pl.ds(_N, 16)]

    def chunk(ci, cc):
        ebase = wid * _ETP + ci * _EC
        pltpu.sync_copy(srcg_hbm.at[pl.ds(ebase, _EC)], src_c)
        pltpu.sync_copy(dstg_hbm.at[pl.ds(ebase, _EC)], dst_c)

        for b in range(2):
            pltpu.async_copy(h_hbm.at[src_c.at[pl.ds(_K * b, _K)]],
                             rows4.at[b], gsem.at[b])

        def exg(t, c):
            sv = src_c[pl.ds(16 * t, 16)]
            dv = dst_c[pl.ds(16 * t, 16)]
            pv = plsc.load_gather(p_v, [sv])
            qv = plsc.load_gather(q_v, [dv])
            a = pv + qv
            e = jnp.where(a >= 0, a, a * 0.01)
            ex_c[pl.ds(16 * t, 16)] = jnp.exp(e - bvec)
            return c
        lax.fori_loop(0, _EC // 16, exg, 0)

        @pl.when(ci == _NC - 1)
        def _():
            for t in range(15):
                ex_c[pl.ds(_EC - 240 + 16 * t, 16)] = zv

        def heavy(g, c):
            eb = g * _K
            b = jnp.bitwise_and(g, _NB - 1)
            bf = jnp.full((16,), b, jnp.int32)

            @pl.when(g + 2 < _GC)
            def _():
                b2 = jnp.bitwise_and(g + 2, _NB - 1)
                eb2 = (g + 2) * _K

                @pl.when(g >= 2)
                def _():
                    ebm = (g - 2) * _K
                    pltpu.make_async_copy(
                        rows4.at[b2],
                        u_sh.at[dst_c.at[pl.ds(ebm, _K)]],
                        ssem.at[b2]).wait()
                    pltpu.make_async_copy(
                        ex_c.at[pl.ds(ebm, _K)],
                        s_sh.at[dst_c.at[pl.ds(ebm, _K)]],
                        ssem.at[b2]).wait()
                pltpu.async_copy(h_hbm.at[src_c.at[pl.ds(eb2, _K)]],
                                 rows4.at[b2], gsem.at[b2])

            pltpu.make_async_copy(h_hbm.at[src_c.at[pl.ds(eb, _K)]],
                                  rows4.at[b], gsem.at[b]).wait()

            for lo, base, nlane in ((0, 0, 16), (16, 16, 16), (24, 32, 8)):
                exv16 = ex_c[pl.ds(eb + lo, 16)]
                lane0 = base - lo

                @plsc.parallel_loop(0, nlane, unroll=2)
                def scale_row(i):
                    exv = exv16[jnp.full((16,), i + lane0, jnp.int32)]
                    jf = jnp.full((16,), base + i, jnp.int32)
                    for c8 in range(8):
                        cvec = iota + 16 * c8
                        x = plsc.load_gather(rows4, [bf, jf, cvec])
                        plsc.store_scatter(rows4, [bf, jf, cvec], x * exv)

            pltpu.async_copy(rows4.at[b], u_sh.at[dst_c.at[pl.ds(eb, _K)]],
                             ssem.at[b], add=True)
            pltpu.async_copy(ex_c.at[pl.ds(eb, _K)],
                             s_sh.at[dst_c.at[pl.ds(eb, _K)]],
                             ssem.at[b], add=True)
            return c
        lax.fori_loop(0, _GC, heavy, 0)

        for k in range(_NB):
            grp = _GC - _NB + k
            b = grp % _NB
            eb = grp * _K
            pltpu.make_async_copy(rows4.at[b],
                                  u_sh.at[dst_c.at[pl.ds(eb, _K)]],
                                  ssem.at[b]).wait()
            pltpu.make_async_copy(ex_c.at[pl.ds(eb, _K)],
                                  s_sh.at[dst_c.at[pl.ds(eb, _K)]],
                                  ssem.at[b]).wait()
        return cc
    lax.fori_loop(0, _NC, chunk, 0)

    plsc.subcore_barrier()

    for k in range(16):
        pltpu.sync_copy(u_sh.at[pl.ds(nbase + 40 * k, 40)], rows4.at[0])
        pltpu.sync_copy(rows4.at[0], u_hbm.at[cid, pl.ds(nbase + 40 * k, 40)])
    for k in range(8):
        pltpu.sync_copy(s_sh.at[pl.ds(nbase + 80 * k, 80)], szbuf)
        pltpu.sync_copy(szbuf,
                        s_hbm.at[pl.ds(cid * _NP + nbase + 80 * k, 80)])


def _div_body(u_ref, st_ref, o_ref):
    num = u_ref[0] + u_ref[1]
    st = st_ref[...]
    den = st[:, 0:1] + st[:, 1:2]
    o_ref[...] = jnp.where(den > 0, num / den, 0.0)


@jax.jit
def kernel(h, edge_index, attn_w):
    w = attn_w[:, 0].reshape(2, _D)

    pq, b = pl.pallas_call(
        _pq_body,
        out_shape=[jax.ShapeDtypeStruct((_N, 2), jnp.float32),
                   jax.ShapeDtypeStruct((8, 128), jnp.float32)],
    )(h, w)

    padw = ((0, 0), (0, _ETP - _E // _NT))
    srcg = jnp.pad(edge_index[0].reshape(_NT, _E // _NT), padw).reshape(-1)
    dstg = jnp.pad(edge_index[1].reshape(_NT, _E // _NT), padw).reshape(-1)
    p_in = jnp.concatenate([pq[:, 0], b[0, :16]])

    mesh = plsc.VectorSubcoreMesh(core_axis_name="c", subcore_axis_name="s",
                                  num_cores=2)
    u2, s2 = pl.kernel(
        _sc_body,
        out_type=[jax.ShapeDtypeStruct((2, _NP, _D), jnp.float32),
                  jax.ShapeDtypeStruct((2 * _NP,), jnp.float32)],
        mesh=mesh,
        compiler_params=pltpu.CompilerParams(needs_layout_passes=False),
        scratch_types=[
            pltpu.VMEM((_CW,), jnp.int32),
            pltpu.VMEM((_CW,), jnp.int32),
            pltpu.VMEM((_CW,), jnp.float32),
            pltpu.VMEM((_N + 16,), jnp.float32),
            pltpu.VMEM((_N,), jnp.float32),
            pltpu.VMEM((_NB, _K, _D), jnp.float32),
            pltpu.VMEM((80,), jnp.float32),
            pltpu.VMEM_SHARED((_NP, _D), jnp.float32),
            pltpu.VMEM_SHARED((_NP,), jnp.float32),
            pltpu.SemaphoreType.DMA((_NB,)),
            pltpu.SemaphoreType.DMA((_NB,)),
        ],
    )(h, srcg, dstg, p_in, pq[:, 1])

    st = s2.reshape(2, _NP).T
    out = pl.pallas_call(
        _div_body,
        grid=(_N // _RB,),
        in_specs=[pl.BlockSpec((2, _RB, _D), lambda i: (0, i, 0)),
                  pl.BlockSpec((_RB, 2), lambda i: (i, 0))],
        out_specs=pl.BlockSpec((_RB, _D), lambda i: (i, 0)),
        out_shape=jax.ShapeDtypeStruct((_N, _D), jnp.float32),
    )(u2, st)
    return out


# speedup vs baseline: 2.3452x; 2.3452x over previous
import jax
import jax.numpy as jnp
from jax import lax
from jax.experimental import pallas as pl
from jax.experimental.pallas import tpu as pltpu
from jax.experimental.pallas import tpu_sc as plsc

_N = 10000
_E = 320000
_D = 128
_NP = 10240
_NT = 32
_ET = _E // _NT
_K = 16
_EC = 2000
_NC = _ET // _EC
_GC = _EC // _K
_NB = 8
_CW = _EC
_RB = 1000


def _pq_body(h_ref, w_ref, pq_ref, b_ref):
    h = h_ref[...]
    w = w_ref[...]
    pq = lax.dot_general(h, w, (((1,), (1,)), ((), ())),
                         preferred_element_type=jnp.float32)
    pq_ref[...] = pq
    s = jnp.max(pq[:, 0]) + jnp.max(pq[:, 1])
    b = jnp.where(s >= 0, s, 0.01 * s)
    b_ref[...] = jnp.full((8, 128), b, jnp.float32)


def _sc_body(h_hbm, srcg_hbm, dstg_hbm, p_hbm, q_hbm, b_hbm,
             u_hbm, s_hbm,
             src_c, dst_c, ex_c, p_v, q_v, b_v, rows8, wbuf, szbuf,
             u_sh, s_sh, gsem, ssem):
    cid = lax.axis_index("c")
    sid = lax.axis_index("s")
    wid = cid * 16 + sid
    nbase = sid * 640

    zv = jnp.zeros((16,), jnp.float32)

    def zrow(r, c):
        for c8 in range(8):
            wbuf[r, pl.ds(16 * c8, 16)] = zv
        return c
    lax.fori_loop(0, 32, zrow, 0)

    def zs(i, c):
        szbuf[pl.ds(16 * i, 16)] = zv
        return c
    lax.fori_loop(0, 40, zs, 0)

    for k in range(20):
        pltpu.sync_copy(wbuf, u_sh.at[pl.ds(nbase + 32 * k, 32)])
    pltpu.sync_copy(szbuf, s_sh.at[pl.ds(nbase, 640)])

    pltpu.sync_copy(p_hbm, p_v)
    pltpu.sync_copy(q_hbm, q_v)
    pltpu.sync_copy(b_hbm.at[0], b_v)

    plsc.subcore_barrier()

    iota = lax.iota(jnp.int32, 16)
    bvec = b_v[pl.ds(0, 16)]

    def chunk(ci, cc):
        ebase = wid * _ET + ci * _EC
        pltpu.sync_copy(srcg_hbm.at[pl.ds(ebase, _EC)], src_c)
        pltpu.sync_copy(dstg_hbm.at[pl.ds(ebase, _EC)], dst_c)

        for b in range(4):
            pltpu.async_copy(h_hbm.at[src_c.at[pl.ds(16 * b, 16)]],
                             rows8.at[b], gsem.at[b])

        def exg(t, c):
            sv = src_c[pl.ds(16 * t, 16)]
            dv = dst_c[pl.ds(16 * t, 16)]
            pv = plsc.load_gather(p_v, [sv])
            qv = plsc.load_gather(q_v, [dv])
            a = pv + qv
            e = jnp.where(a >= 0, a, a * 0.01)
            ex_c[pl.ds(16 * t, 16)] = jnp.exp(e - bvec)
            return c
        lax.fori_loop(0, _GC, exg, 0)

        def heavy(g, c):
            eb = g * _K
            b = jnp.bitwise_and(g, _NB - 1)
            bf = jnp.full((16,), b, jnp.int32)

            @pl.when(g + 4 < _GC)
            def _():
                b4 = jnp.bitwise_and(g + 4, _NB - 1)
                eb4 = (g + 4) * _K

                @pl.when(g >= 4)
                def _():
                    ebm = (g - 4) * _K
                    pltpu.make_async_copy(
                        rows8.at[b4],
                        u_sh.at[dst_c.at[pl.ds(ebm, _K)]],
                        ssem.at[b4]).wait()
                    pltpu.make_async_copy(
                        ex_c.at[pl.ds(ebm, _K)],
                        s_sh.at[dst_c.at[pl.ds(ebm, _K)]],
                        ssem.at[b4]).wait()
                pltpu.async_copy(h_hbm.at[src_c.at[pl.ds(eb4, _K)]],
                                 rows8.at[b4], gsem.at[b4])

            pltpu.make_async_copy(h_hbm.at[src_c.at[pl.ds(eb, _K)]],
                                  rows8.at[b], gsem.at[b]).wait()

            exv16 = ex_c[pl.ds(eb, 16)]

            @plsc.parallel_loop(0, 16, unroll=2)
            def scale_row(i):
                exv = exv16[jnp.full((16,), i, jnp.int32)]
                jf = jnp.full((16,), i, jnp.int32)
                for c8 in range(8):
                    cvec = iota + 16 * c8
                    x = plsc.load_gather(rows8, [bf, jf, cvec])
                    plsc.store_scatter(rows8, [bf, jf, cvec], x * exv)

            pltpu.async_copy(rows8.at[b], u_sh.at[dst_c.at[pl.ds(eb, _K)]],
                             ssem.at[b], add=True)
            pltpu.async_copy(ex_c.at[pl.ds(eb, _K)],
                             s_sh.at[dst_c.at[pl.ds(eb, _K)]],
                             ssem.at[b], add=True)
            return c
        lax.fori_loop(0, _GC, heavy, 0)

        for k in range(_NB):
            grp = _GC - _NB + k
            b = grp % _NB
            eb = grp * _K
            pltpu.make_async_copy(rows8.at[b],
                                  u_sh.at[dst_c.at[pl.ds(eb, _K)]],
                                  ssem.at[b]).wait()
            pltpu.make_async_copy(ex_c.at[pl.ds(eb, _K)],
                                  s_sh.at[dst_c.at[pl.ds(eb, _K)]],
                                  ssem.at[b]).wait()
        return cc
    lax.fori_loop(0, _NC, chunk, 0)

    plsc.subcore_barrier()

    for k in range(20):
        pltpu.sync_copy(u_sh.at[pl.ds(nbase + 32 * k, 32)], wbuf)
        pltpu.sync_copy(wbuf, u_hbm.at[cid, pl.ds(nbase + 32 * k, 32)])
    pltpu.sync_copy(s_sh.at[pl.ds(nbase, 640)], szbuf)
    pltpu.sync_copy(szbuf, s_hbm.at[cid, pl.ds(nbase, 640)])


def _div_body(u_ref, st_ref, o_ref):
    num = u_ref[0] + u_ref[1]
    st = st_ref[...]
    den = st[:, 0:1] + st[:, 1:2]
    o_ref[...] = jnp.where(den > 0, num / den, 0.0)


@jax.jit
def kernel(h, edge_index, attn_w):
    w = attn_w[:, 0].reshape(2, _D)

    pq, b = pl.pallas_call(
        _pq_body,
        out_shape=[jax.ShapeDtypeStruct((_N, 2), jnp.float32),
                   jax.ShapeDtypeStruct((8, 128), jnp.float32)],
    )(h, w)

    srcg = edge_index[0]
    dstg = edge_index[1]

    mesh = plsc.VectorSubcoreMesh(core_axis_name="c", subcore_axis_name="s",
                                  num_cores=2)
    u2, s2 = pl.kernel(
        _sc_body,
        out_type=[jax.ShapeDtypeStruct((2, _NP, _D), jnp.float32),
                  jax.ShapeDtypeStruct((2, _NP), jnp.float32)],
        mesh=mesh,
        compiler_params=pltpu.CompilerParams(needs_layout_passes=False),
        scratch_types=[
            pltpu.VMEM((_CW,), jnp.int32),
            pltpu.VMEM((_CW,), jnp.int32),
            pltpu.VMEM((_CW,), jnp.float32),
            pltpu.VMEM((_N,), jnp.float32),
            pltpu.VMEM((_N,), jnp.float32),
            pltpu.VMEM((128,), jnp.float32),
            pltpu.VMEM((_NB, _K, _D), jnp.float32),
            pltpu.VMEM((32, _D), jnp.float32),
            pltpu.VMEM((640,), jnp.float32),
            pltpu.VMEM_SHARED((_NP, _D), jnp.float32),
            pltpu.VMEM_SHARED((_NP,), jnp.float32),
            pltpu.SemaphoreType.DMA((_NB,)),
            pltpu.SemaphoreType.DMA((_NB,)),
        ],
    )(h, srcg, dstg, pq[:, 0], pq[:, 1], b)

    st = s2.T
    out = pl.pallas_call(
        _div_body,
        grid=(_N // _RB,),
        in_specs=[pl.BlockSpec((2, _RB, _D), lambda i: (0, i, 0)),
                  pl.BlockSpec((_RB, 2), lambda i: (i, 0))],
        out_specs=pl.BlockSpec((_RB, _D), lambda i: (i, 0)),
        out_shape=jax.ShapeDtypeStruct((_N, _D), jnp.float32),
    )(u2, st)
    return out
